# baseline (speedup 1.0000x reference)
import functools

import jax
import jax.numpy as jnp
from jax.experimental import pallas as pl
from jax.experimental.pallas import tpu as pltpu

NDIMS = 16
BB = 32


def _mm_block(x_ref, wt_ref, o_ref):
    o_ref[...] = jax.lax.dot_general(
        x_ref[...].astype(jnp.bfloat16), wt_ref[...].astype(jnp.bfloat16),
        dimension_numbers=(((1,), (0,)), ((), ())),
        preferred_element_type=jnp.float32,
    )


@functools.partial(jax.jit, static_argnames=())
def kernel(x, weights):
    n = weights.shape[0]
    b = x.shape[0]
    wt = weights.T
    grid = (pl.cdiv(b, BB),)
    return pl.pallas_call(
        _mm_block,
        grid=grid,
        in_specs=[
            pl.BlockSpec((BB, NDIMS), lambda i: (i, 0)),
            pl.BlockSpec((NDIMS, n), lambda i: (0, 0)),
        ],
        out_specs=pl.BlockSpec((BB, n), lambda i: (i, 0)),
        out_shape=jax.ShapeDtypeStruct((b, n), jnp.float32),
        compiler_params=pltpu.CompilerParams(
            dimension_semantics=("arbitrary",),
        ),
    )(x, wt)


# speedup vs baseline: 1.0054x; 1.0054x over previous
import functools

import jax
import jax.numpy as jnp
from jax.experimental import pallas as pl
from jax.experimental.pallas import tpu as pltpu

NDIMS = 16
BB = 32
NBUF = 2
NSPLIT = 8
ROWS = BB // NSPLIT


def _mm_block(x_ref, wt_ref, o_hbm, acc_ref, sems):
    i = pl.program_id(0)
    nsteps = pl.num_programs(0)
    buf = jax.lax.rem(i, NBUF)

    @pl.when(i >= NBUF)
    def _():
        for s in range(NSPLIT):
            pltpu.make_async_copy(
                acc_ref.at[buf, s * ROWS:(s + 1) * ROWS, :],
                o_hbm.at[pl.ds((i - NBUF) * BB + s * ROWS, ROWS), :],
                sems.at[buf, s],
            ).wait()

    acc_ref[buf] = jax.lax.dot_general(
        x_ref[...].astype(jnp.bfloat16), wt_ref[...].astype(jnp.bfloat16),
        dimension_numbers=(((1,), (0,)), ((), ())),
        preferred_element_type=jnp.float32,
    )

    for s in range(NSPLIT):
        pltpu.make_async_copy(
            acc_ref.at[buf, s * ROWS:(s + 1) * ROWS, :],
            o_hbm.at[pl.ds(i * BB + s * ROWS, ROWS), :],
            sems.at[buf, s],
        ).start()

    @pl.when(i == nsteps - 1)
    def _():
        for b in range(NBUF):
            step = i - ((i - b) % NBUF)
            for s in range(NSPLIT):
                pltpu.make_async_copy(
                    acc_ref.at[b, s * ROWS:(s + 1) * ROWS, :],
                    o_hbm.at[pl.ds(step * BB + s * ROWS, ROWS), :],
                    sems.at[b, s],
                ).wait()


@functools.partial(jax.jit, static_argnames=())
def kernel(x, weights):
    n = weights.shape[0]
    b = x.shape[0]
    wt = weights.T
    grid = (b // BB,)
    return pl.pallas_call(
        _mm_block,
        grid=grid,
        in_specs=[
            pl.BlockSpec((BB, NDIMS), lambda i: (i, 0)),
            pl.BlockSpec((NDIMS, n), lambda i: (0, 0)),
        ],
        out_specs=pl.BlockSpec(memory_space=pl.ANY),
        out_shape=jax.ShapeDtypeStruct((b, n), jnp.float32),
        scratch_shapes=[
            pltpu.VMEM((NBUF, BB, n), jnp.float32),
            pltpu.SemaphoreType.DMA((NBUF, NSPLIT)),
        ],
        compiler_params=pltpu.CompilerParams(
            dimension_semantics=("arbitrary",),
        ),
    )(x, wt)
